# Initial kernel scaffold; baseline (speedup 1.0000x reference)
import jax
import jax.numpy as jnp
from jax.experimental import pallas as pl


def kernel(feature_indices, feature_values, weight, bias):
    raise NotImplementedError("write your pallas kernel here")


# speedup vs baseline: 6.3744x; 6.3744x over previous
import jax
import jax.numpy as jnp
from jax import lax
from jax.experimental import pallas as pl
from jax.experimental.pallas import tpu as pltpu
from jax.experimental.pallas import tpu_sc as plsc

B = 16384
L = 50
V = 100000
D = 128
LP = 64

NC = 2
NS = 16
NW = NC * NS
BPW = B // NW
PAIR = 2
NPAIR = BPW // PAIR
OUTCH = 64
PAIRS_PER_OUTCH = OUTCH // PAIR
NOUTCH = BPW // OUTCH


def _sc_body(idx_hbm, vals_hbm, w_hbm, bias_hbm, out_hbm,
             idx_v, vals_v, bias_v, rows_v, out_v, sem_g):
    wid = lax.axis_index("s") * NC + lax.axis_index("c")
    base = wid * BPW

    pltpu.sync_copy(idx_hbm.at[pl.ds(wid * NPAIR, NPAIR)], idx_v)
    pltpu.sync_copy(vals_hbm.at[pl.ds(base, BPW)], vals_v)
    pltpu.sync_copy(bias_hbm, bias_v)

    def outch_body(oc, carry):
        def pair_body(t2, carry2):
            t = oc * PAIRS_PER_OUTCH + t2
            pltpu.async_copy(w_hbm.at[idx_v.at[t]], rows_v, sem_g).wait()
            for p in range(PAIR):
                rr = t * PAIR + p
                rloc = t2 * PAIR + p
                accs = [bias_v[pl.ds(d8 * 16, 16)] for d8 in range(8)]
                for k16 in range((L + 15) // 16):
                    vv = vals_v[rr, pl.ds(k16 * 16, 16)]
                    for j in range(min(16, L - k16 * 16)):
                        bk = vv[j]
                        row = p * L + k16 * 16 + j
                        for d8 in range(8):
                            accs[d8] = accs[d8] + rows_v[row, pl.ds(d8 * 16, 16)] * bk
                for d8 in range(8):
                    out_v[rloc, pl.ds(d8 * 16, 16)] = accs[d8]
            return carry2
        lax.fori_loop(0, PAIRS_PER_OUTCH, pair_body, carry)
        pltpu.sync_copy(out_v, out_hbm.at[pl.ds(base + oc * OUTCH, OUTCH)])
        return carry

    lax.fori_loop(0, NOUTCH, outch_body, 0)


def kernel(feature_indices, feature_values, weight, bias):
    idx2 = feature_indices.reshape(B // PAIR, L * PAIR)
    vals_p = jnp.pad(feature_values, ((0, 0), (0, LP - L)))
    mesh = plsc.VectorSubcoreMesh(core_axis_name="c", subcore_axis_name="s")
    run = pl.kernel(
        _sc_body,
        out_type=jax.ShapeDtypeStruct((B, D), jnp.float32),
        mesh=mesh,
        scratch_types=[
            pltpu.VMEM((NPAIR, L * PAIR), jnp.int32),
            pltpu.VMEM((BPW, LP), jnp.float32),
            pltpu.VMEM((D,), jnp.float32),
            pltpu.VMEM((L * PAIR, D), jnp.float32),
            pltpu.VMEM((OUTCH, D), jnp.float32),
            pltpu.SemaphoreType.DMA,
        ],
    )
    return run(idx2, vals_p, weight, bias)


# speedup vs baseline: 9.2319x; 1.4483x over previous
import jax
import jax.numpy as jnp
from jax import lax
from jax.experimental import pallas as pl
from jax.experimental.pallas import tpu as pltpu
from jax.experimental.pallas import tpu_sc as plsc

B = 16384
L = 50
V = 100000
D = 128
LP = 64

NC = 2
NS = 16
NW = NC * NS
BPW = B // NW
OUTCH = 32
ROWS_PER_OUTCH = OUTCH
NOUTCH = BPW // OUTCH


def _sc_body(idx_hbm, vals_hbm, w_hbm, bias_hbm, out_hbm,
             idx_v, vals_v, bias_v, rows_v0, rows_v1, out_v, sem0, sem1):
    wid = lax.axis_index("s") * NC + lax.axis_index("c")
    base = wid * BPW

    pltpu.sync_copy(idx_hbm.at[pl.ds(base, BPW)], idx_v)
    pltpu.sync_copy(bias_hbm, bias_v)

    def gather_start(t, buf, sem):
        pltpu.async_copy(w_hbm.at[idx_v.at[t]], buf, sem)

    def gather_wait(buf, sem):
        pltpu.make_async_copy(w_hbm.at[idx_v.at[0]], buf, sem).wait()

    def compute_row(t, t_loc, buf):
        accs = [bias_v[pl.ds(d8 * 16, 16)] for d8 in range(8)]
        for k16 in range((L + 15) // 16):
            vv = vals_v[t_loc, pl.ds(k16 * 16, 16)]
            for j in range(min(16, L - k16 * 16)):
                bk = vv[j]
                row = k16 * 16 + j
                for d8 in range(8):
                    accs[d8] = accs[d8] + buf[row, pl.ds(d8 * 16, 16)] * bk
        for d8 in range(8):
            out_v[t_loc, pl.ds(d8 * 16, 16)] = accs[d8]

    gather_start(0, rows_v0, sem0)

    def outch_body(oc, carry):
        pltpu.sync_copy(vals_hbm.at[pl.ds(base + oc * OUTCH, OUTCH)], vals_v)

        def u_body(u2, carry2):
            t0 = oc * ROWS_PER_OUTCH + u2 * 2
            gather_start(t0 + 1, rows_v1, sem1)
            gather_wait(rows_v0, sem0)
            compute_row(t0, u2 * 2, rows_v0)
            tn = jnp.minimum(t0 + 2, BPW - 1)
            gather_start(tn, rows_v0, sem0)
            gather_wait(rows_v1, sem1)
            compute_row(t0 + 1, u2 * 2 + 1, rows_v1)
            return carry2
        lax.fori_loop(0, ROWS_PER_OUTCH // 2, u_body, carry)
        pltpu.sync_copy(out_v, out_hbm.at[pl.ds(base + oc * OUTCH, OUTCH)])
        return carry

    lax.fori_loop(0, NOUTCH, outch_body, 0)
    gather_wait(rows_v0, sem0)


def kernel(feature_indices, feature_values, weight, bias):
    vals_p = jnp.pad(feature_values, ((0, 0), (0, LP - L)))
    mesh = plsc.VectorSubcoreMesh(core_axis_name="c", subcore_axis_name="s")
    run = pl.kernel(
        _sc_body,
        out_type=jax.ShapeDtypeStruct((B, D), jnp.float32),
        mesh=mesh,
        scratch_types=[
            pltpu.VMEM((BPW, L), jnp.int32),
            pltpu.VMEM((OUTCH, LP), jnp.float32),
            pltpu.VMEM((D,), jnp.float32),
            pltpu.VMEM((L, D), jnp.float32),
            pltpu.VMEM((L, D), jnp.float32),
            pltpu.VMEM((OUTCH, D), jnp.float32),
            pltpu.SemaphoreType.DMA,
            pltpu.SemaphoreType.DMA,
        ],
    )
    return run(feature_indices, vals_p, weight, bias)
